# Initial kernel scaffold; baseline (speedup 1.0000x reference)
import jax
import jax.numpy as jnp
from jax.experimental import pallas as pl


def kernel(x, word_ids, fixed_L, fixed_x0, fixed_k, random_x0):
    raise NotImplementedError("write your pallas kernel here")


# speedup vs baseline: 166.8300x; 166.8300x over previous
import functools

import jax
import jax.numpy as jnp
from jax import lax
from jax.experimental import pallas as pl
from jax.experimental.pallas import tpu as pltpu
from jax.experimental.pallas import tpu_sc as plsc

N_OBS = 3276800
NUM_CORES = 2
NUM_SUBCORES = 16
NUM_WORKERS = NUM_CORES * NUM_SUBCORES
PER_WORKER = N_OBS // NUM_WORKERS
CHUNK = 12800
NUM_CHUNKS = PER_WORKER // CHUNK
LANES = 16


def _sc_logistic(x, word_ids, l_b, x0_b, nk_b, random_x0):
    mesh = plsc.VectorSubcoreMesh(core_axis_name="c", subcore_axis_name="s")

    @functools.partial(
        pl.kernel,
        out_type=jax.ShapeDtypeStruct((N_OBS,), jnp.float32),
        mesh=mesh,
        scratch_types=[
            pltpu.VMEM((LANES,), jnp.float32),
            pltpu.VMEM((LANES,), jnp.float32),
            pltpu.VMEM((LANES,), jnp.float32),
            pltpu.VMEM((CHUNK,), jnp.int32),
            pltpu.VMEM((CHUNK,), jnp.float32),
            pltpu.VMEM((CHUNK,), jnp.float32),
            pltpu.VMEM((CHUNK,), jnp.float32),
            pltpu.SemaphoreType.DMA,
        ],
    )
    def run(x_hbm, ids_hbm, l_hbm, x0_hbm, nk_hbm, table_hbm, out_hbm,
            l_v, x0_v, nk_v, idx_v, x_v, val_v, o_v, sem):
        wid = lax.axis_index("s") * NUM_CORES + lax.axis_index("c")
        base = wid * PER_WORKER
        pltpu.sync_copy(l_hbm, l_v)
        pltpu.sync_copy(x0_hbm, x0_v)
        pltpu.sync_copy(nk_hbm, nk_v)
        lv = l_v[...]
        x0v = x0_v[...]
        nkv = nk_v[...]

        @pl.loop(0, NUM_CHUNKS)
        def _chunk(c):
            off = base + c * CHUNK
            pltpu.sync_copy(ids_hbm.at[pl.ds(off, CHUNK)], idx_v)
            pltpu.sync_copy(x_hbm.at[pl.ds(off, CHUNK)], x_v)
            pltpu.async_copy(table_hbm.at[idx_v], val_v, sem).wait()

            @pl.loop(0, CHUNK // LANES)
            def _vec(i):
                s = pl.ds(i * LANES, LANES)
                arg = nkv * (x_v[s] - x0v - val_v[s])
                o_v[s] = lv / (1.0 + jnp.exp(arg))

            pltpu.sync_copy(o_v, out_hbm.at[pl.ds(off, CHUNK)])

    return run(x, word_ids, l_b, x0_b, nk_b, random_x0)


def kernel(x, word_ids, fixed_L, fixed_x0, fixed_k, random_x0):
    ids = word_ids.astype(jnp.int32)
    l_b = jnp.broadcast_to(jnp.asarray(fixed_L, jnp.float32), (LANES,))
    x0_b = jnp.broadcast_to(jnp.asarray(fixed_x0, jnp.float32), (LANES,))
    nk_b = jnp.broadcast_to(-jnp.asarray(fixed_k, jnp.float32), (LANES,))
    return _sc_logistic(x, ids, l_b, x0_b, nk_b, random_x0)


# speedup vs baseline: 197.3921x; 1.1832x over previous
import functools

import jax
import jax.numpy as jnp
from jax import lax
from jax.experimental import pallas as pl
from jax.experimental.pallas import tpu as pltpu
from jax.experimental.pallas import tpu_sc as plsc

N_OBS = 3276800
NUM_CORES = 2
NUM_SUBCORES = 16
NUM_WORKERS = NUM_CORES * NUM_SUBCORES
PER_WORKER = N_OBS // NUM_WORKERS
CHUNK = 12800
NUM_CHUNKS = PER_WORKER // CHUNK
LANES = 16
NBUF = 2


def _sc_logistic(x, word_ids, a_b, b_b, l_b, random_x0):
    mesh = plsc.VectorSubcoreMesh(core_axis_name="c", subcore_axis_name="s")

    scratch = [
        pltpu.VMEM((LANES,), jnp.float32),
        pltpu.VMEM((LANES,), jnp.float32),
        pltpu.VMEM((LANES,), jnp.float32),
    ]
    for _ in range(NBUF):
        scratch += [
            pltpu.VMEM((CHUNK,), jnp.int32),
            pltpu.VMEM((CHUNK,), jnp.float32),
            pltpu.VMEM((CHUNK,), jnp.float32),
            pltpu.VMEM((CHUNK,), jnp.float32),
            pltpu.SemaphoreType.DMA,
            pltpu.SemaphoreType.DMA,
        ]

    @functools.partial(
        pl.kernel,
        out_type=jax.ShapeDtypeStruct((N_OBS,), jnp.float32),
        mesh=mesh,
        scratch_types=scratch,
    )
    def run(x_hbm, ids_hbm, a_hbm, b_hbm, l_hbm, table_hbm, out_hbm,
            a_v, b_v, l_v, *bufs):
        idx_v = [bufs[6 * i + 0] for i in range(NBUF)]
        x_v = [bufs[6 * i + 1] for i in range(NBUF)]
        val_v = [bufs[6 * i + 2] for i in range(NBUF)]
        o_v = [bufs[6 * i + 3] for i in range(NBUF)]
        gsem = [bufs[6 * i + 4] for i in range(NBUF)]
        wsem = [bufs[6 * i + 5] for i in range(NBUF)]

        wid = lax.axis_index("s") * NUM_CORES + lax.axis_index("c")
        base = wid * PER_WORKER
        pltpu.sync_copy(a_hbm, a_v)
        pltpu.sync_copy(b_hbm, b_v)
        pltpu.sync_copy(l_hbm, l_v)
        av = a_v[...]
        bv = b_v[...]
        lv = l_v[...]

        def stage(c):
            bb = c % NBUF
            off = base + c * CHUNK
            pltpu.sync_copy(ids_hbm.at[pl.ds(off, CHUNK)], idx_v[bb])
            pltpu.sync_copy(x_hbm.at[pl.ds(off, CHUNK)], x_v[bb])
            return pltpu.async_copy(table_hbm.at[idx_v[bb]], val_v[bb], gsem[bb])

        gathers = {}
        writes = {}
        gathers[0] = stage(0)
        for c in range(NUM_CHUNKS):
            bb = c % NBUF
            if c + 1 < NUM_CHUNKS:
                gathers[c + 1] = stage(c + 1)
            gathers[c].wait()
            if c - NBUF >= 0:
                writes[c - NBUF].wait()

            @plsc.parallel_loop(0, CHUNK, LANES)
            def _vec(i):
                s = pl.ds(i, LANES)
                arg = av * (x_v[bb][s] - val_v[bb][s]) + bv
                o_v[bb][s] = lv / (1.0 + jnp.exp(arg))

            off = base + c * CHUNK
            writes[c] = pltpu.async_copy(
                o_v[bb], out_hbm.at[pl.ds(off, CHUNK)], wsem[bb])
        for c in range(max(0, NUM_CHUNKS - NBUF), NUM_CHUNKS):
            writes[c].wait()

    return run(x, word_ids, a_b, b_b, l_b, random_x0)


def kernel(x, word_ids, fixed_L, fixed_x0, fixed_k, random_x0):
    ids = word_ids.astype(jnp.int32)
    k = jnp.asarray(fixed_k, jnp.float32)
    x0 = jnp.asarray(fixed_x0, jnp.float32)
    a_b = jnp.broadcast_to(-k, (LANES,))
    b_b = jnp.broadcast_to(k * x0, (LANES,))
    l_b = jnp.broadcast_to(jnp.asarray(fixed_L, jnp.float32), (LANES,))
    return _sc_logistic(x, ids, a_b, b_b, l_b, random_x0)


# speedup vs baseline: 452.0882x; 2.2903x over previous
import functools

import jax
import jax.numpy as jnp
from jax import lax
from jax.experimental import pallas as pl
from jax.experimental.pallas import tpu as pltpu
from jax.experimental.pallas import tpu_sc as plsc

N_OBS = 3276800
TABLE = 1000000
NUM_CORES = 2
NUM_SUBCORES = 16
NUM_WORKERS = NUM_CORES * NUM_SUBCORES
PER_WORKER = N_OBS // NUM_WORKERS
CHUNK = 6400
NUM_CHUNKS = PER_WORKER // CHUNK
LANES = 16
NBUF_IN = 3
NBUF = 2
NPIECES = -(-TABLE // (NUM_SUBCORES * CHUNK))


def _sc_logistic(x, word_ids, a_b, b_b, l_b, random_x0):
    mesh = plsc.VectorSubcoreMesh(core_axis_name="c", subcore_axis_name="s")

    scratch = [
        pltpu.VMEM_SHARED((TABLE,), jnp.float32),
        pltpu.SemaphoreType.DMA,
        pltpu.VMEM((LANES,), jnp.float32),
        pltpu.VMEM((LANES,), jnp.float32),
        pltpu.VMEM((LANES,), jnp.float32),
    ]
    for _ in range(NBUF_IN):
        scratch += [
            pltpu.VMEM((CHUNK,), jnp.int32),
            pltpu.VMEM((CHUNK,), jnp.float32),
            pltpu.SemaphoreType.DMA,
        ]
    for _ in range(NBUF):
        scratch += [
            pltpu.VMEM((CHUNK,), jnp.float32),
            pltpu.VMEM((CHUNK,), jnp.float32),
            pltpu.SemaphoreType.DMA,
            pltpu.SemaphoreType.DMA,
        ]

    @functools.partial(
        pl.kernel,
        out_type=jax.ShapeDtypeStruct((N_OBS,), jnp.float32),
        mesh=mesh,
        scratch_types=scratch,
    )
    def run(x_hbm, ids_hbm, a_hbm, b_hbm, l_hbm, table_hbm, out_hbm,
            tbl_sp, tsem, a_v, b_v, l_v, *bufs):
        idx_v = [bufs[3 * i + 0] for i in range(NBUF_IN)]
        x_v = [bufs[3 * i + 1] for i in range(NBUF_IN)]
        isem = [bufs[3 * i + 2] for i in range(NBUF_IN)]
        gb = bufs[3 * NBUF_IN:]
        val_v = [gb[4 * i + 0] for i in range(NBUF)]
        o_v = [gb[4 * i + 1] for i in range(NBUF)]
        gsem = [gb[4 * i + 2] for i in range(NBUF)]
        wsem = [gb[4 * i + 3] for i in range(NBUF)]

        sid = lax.axis_index("s")
        wid = sid * NUM_CORES + lax.axis_index("c")
        base = wid * PER_WORKER

        pltpu.sync_copy(a_hbm, a_v)
        pltpu.sync_copy(b_hbm, b_v)
        pltpu.sync_copy(l_hbm, l_v)
        av = a_v[...]
        bv = b_v[...]
        lv = l_v[...]

        def stage_in(c):
            bi = c % NBUF_IN
            off = base + c * CHUNK
            h1 = pltpu.async_copy(
                ids_hbm.at[pl.ds(off, CHUNK)], idx_v[bi], isem[bi])
            h2 = pltpu.async_copy(
                x_hbm.at[pl.ds(off, CHUNK)], x_v[bi], isem[bi])
            return (h1, h2)

        ins = {}
        for c in range(min(NBUF_IN, NUM_CHUNKS)):
            ins[c] = stage_in(c)

        def tload(p):
            poff = jnp.minimum((p * NUM_SUBCORES + sid) * CHUNK,
                               TABLE - CHUNK)
            return poff, pltpu.async_copy(
                table_hbm.at[pl.ds(poff, CHUNK)], val_v[p % NBUF],
                gsem[p % NBUF])

        tloads = {0: tload(0)}
        tstores = {}
        for p in range(NPIECES):
            poff, h = tloads[p]
            h.wait()
            tstores[p] = pltpu.async_copy(
                val_v[p % NBUF], tbl_sp.at[pl.ds(poff, CHUNK)], tsem)
            if p + 1 < NPIECES:
                if p - 1 >= 0:
                    tstores[p - 1].wait()
                tloads[p + 1] = tload(p + 1)
        tstores[NPIECES - 1].wait()
        if NPIECES >= 2:
            tstores[NPIECES - 2].wait()
        plsc.subcore_barrier()

        def gather(c):
            bi = c % NBUF_IN
            bg = c % NBUF
            return pltpu.async_copy(
                tbl_sp.at[idx_v[bi]], val_v[bg], gsem[bg])

        gathers = {}
        writes = {}
        ins[0][0].wait()
        ins[0][1].wait()
        gathers[0] = gather(0)

        for c in range(NUM_CHUNKS):
            bb = c % NBUF
            if c + 1 < NUM_CHUNKS:
                ins[c + 1][0].wait()
                ins[c + 1][1].wait()
                gathers[c + 1] = gather(c + 1)
            gathers[c].wait()
            if c - NBUF >= 0:
                writes[c - NBUF].wait()

            bi = c % NBUF_IN

            @plsc.parallel_loop(0, CHUNK, LANES)
            def _vec(i):
                s = pl.ds(i, LANES)
                arg = av * (x_v[bi][s] - val_v[bb][s]) + bv
                o_v[bb][s] = lv / (1.0 + jnp.exp(arg))

            off = base + c * CHUNK
            writes[c] = pltpu.async_copy(
                o_v[bb], out_hbm.at[pl.ds(off, CHUNK)], wsem[bb])
            if c + NBUF_IN < NUM_CHUNKS:
                ins[c + NBUF_IN] = stage_in(c + NBUF_IN)

        for c in range(max(0, NUM_CHUNKS - NBUF), NUM_CHUNKS):
            writes[c].wait()

    return run(x, word_ids, a_b, b_b, l_b, random_x0)


def kernel(x, word_ids, fixed_L, fixed_x0, fixed_k, random_x0):
    ids = word_ids.astype(jnp.int32)
    k = jnp.asarray(fixed_k, jnp.float32)
    x0 = jnp.asarray(fixed_x0, jnp.float32)
    a_b = jnp.broadcast_to(-k, (LANES,))
    b_b = jnp.broadcast_to(k * x0, (LANES,))
    l_b = jnp.broadcast_to(jnp.asarray(fixed_L, jnp.float32), (LANES,))
    return _sc_logistic(x, ids, a_b, b_b, l_b, random_x0)
